# baseline (speedup 1.0000x reference)
import jax
import jax.numpy as jnp
from jax.experimental import pallas as pl
from jax.experimental.pallas import tpu as pltpu

_N = 10000
_IN_F = 128
_HID = 64
_OUT = 64
_ALPHA = 0.2
_NEGBIG = -9e30

_BI = 1024
_BJ = 1024
_NI = (_N + _BI - 1) // _BI
_NJ = (_N + _BJ - 1) // _BJ
_BG = _BI // 8
_NP = _NI * _BG
_NC = _NJ * _BJ


def _lrelu(z):
    return jnp.where(z >= 0, z, _ALPHA * z)


def _row_t(a2, wh):
    return jax.lax.dot_general(a2, wh, (((0,), (1,)), ((), ())),
                               preferred_element_type=jnp.float32)


def _prelude_kernel(x_ref, wlin_ref, blin_ref, w0_ref, w1_ref,
                    a01_ref, a02_ref, a11_ref, a12_ref,
                    wh0_o, wh1_o, s0_o, s1_o, t0_o, t1_o, t0m_o, t1m_o,
                    t0m_s, t1m_s):
    i = pl.program_id(0)
    h = jnp.dot(x_ref[...], wlin_ref[...],
                preferred_element_type=jnp.float32) + blin_ref[...]
    wh0 = jnp.dot(h, w0_ref[...], preferred_element_type=jnp.float32)
    wh1 = jnp.dot(h, w1_ref[...], preferred_element_type=jnp.float32)
    wh0_o[...] = wh0
    wh1_o[...] = wh1
    s0_o[...] = jnp.dot(wh0, a01_ref[...], preferred_element_type=jnp.float32)
    s1_o[...] = jnp.dot(wh1, a11_ref[...], preferred_element_type=jnp.float32)
    t0 = _row_t(a02_ref[...], wh0)
    t1 = _row_t(a12_ref[...], wh1)
    t0_o[...] = t0
    t1_o[...] = t1
    colv = (i * _BI + jax.lax.broadcasted_iota(jnp.int32, (1, _BI), 1)) < _N
    bm0 = jnp.max(jnp.where(colv, t0, _NEGBIG), keepdims=True)
    bm1 = jnp.max(jnp.where(colv, t1, _NEGBIG), keepdims=True)
    prev0 = jnp.where(i == 0, jnp.full((1, 1), _NEGBIG), t0m_s[...])
    prev1 = jnp.where(i == 0, jnp.full((1, 1), _NEGBIG), t1m_s[...])
    t0m_s[...] = jnp.maximum(prev0, bm0)
    t1m_s[...] = jnp.maximum(prev1, bm1)

    @pl.when(i == _NI - 1)
    def _():
        t0m_o[...] = t0m_s[...]
        t1m_o[...] = t1m_s[...]


def _pass1_kernel(adj_ref, wh0_ref, wh1_ref, s0_ref, s1_ref, t0_ref, t1_ref,
                  t0m_ref, t1m_ref, wend_ref, ae1_ref, ae2_ref,
                  whe_o, se_o, te_o, tem_o, pk_o,
                  acc0, acc1, l0, l1, tem_s):
    i = pl.program_id(0)
    j = pl.program_id(1)

    @pl.when(j == 0)
    def _():
        acc0[...] = jnp.zeros_like(acc0)
        acc1[...] = jnp.zeros_like(acc1)
        l0[...] = jnp.zeros_like(l0)
        l1[...] = jnp.zeros_like(l1)

    adjb = adj_ref[...] > 0
    colv = (j * _BJ + jax.lax.broadcasted_iota(jnp.int32, (1, _BJ), 1)) < _N
    ok = jnp.logical_and(adjb, colv)
    rowv_j = (j * _BJ + jax.lax.broadcasted_iota(jnp.int32, (_BJ, 1), 0)) < _N
    wh0 = jnp.where(rowv_j, wh0_ref[...], 0.0).astype(jnp.bfloat16)
    wh1 = jnp.where(rowv_j, wh1_ref[...], 0.0).astype(jnp.bfloat16)

    oki = ok.astype(jnp.int32)
    pk = oki[0:_BG, :]
    for r in range(1, 8):
        pk += oki[r * _BG:(r + 1) * _BG, :] << r
    pk_o[...] = pk.astype(jnp.uint8)

    m0 = _lrelu(s0_ref[...] + t0m_ref[...])
    e0 = _lrelu(s0_ref[...] + t0_ref[...])
    p0 = jnp.where(ok, jnp.exp(e0 - m0), 0.0)
    l0[...] += jnp.sum(p0, axis=1, keepdims=True)
    acc0[...] += jnp.dot(p0.astype(jnp.bfloat16), wh0,
                         preferred_element_type=jnp.float32)

    m1 = _lrelu(s1_ref[...] + t1m_ref[...])
    e1 = _lrelu(s1_ref[...] + t1_ref[...])
    p1 = jnp.where(ok, jnp.exp(e1 - m1), 0.0)
    l1[...] += jnp.sum(p1, axis=1, keepdims=True)
    acc1[...] += jnp.dot(p1.astype(jnp.bfloat16), wh1,
                         preferred_element_type=jnp.float32)

    @pl.when(j == _NJ - 1)
    def _():
        h0 = acc0[...] / jnp.maximum(l0[...], 1e-30)
        h1 = acc1[...] / jnp.maximum(l1[...], 1e-30)
        x0 = jnp.where(h0 > 0, h0, jnp.exp(h0) - 1.0)
        x1 = jnp.where(h1 > 0, h1, jnp.exp(h1) - 1.0)
        whe = (jnp.dot(x0, wend_ref[:_HID, :], preferred_element_type=jnp.float32)
               + jnp.dot(x1, wend_ref[_HID:, :], preferred_element_type=jnp.float32))
        whe_o[...] = whe
        se_o[...] = jnp.dot(whe, ae1_ref[...], preferred_element_type=jnp.float32)
        te = _row_t(ae2_ref[...], whe)
        te_o[...] = te
        colv_i = (i * _BI + jax.lax.broadcasted_iota(jnp.int32, (1, _BI), 1)) < _N
        bm = jnp.max(jnp.where(colv_i, te, _NEGBIG), keepdims=True)
        prev = jnp.where(i == 0, jnp.full((1, 1), _NEGBIG), tem_s[...])
        tem_s[...] = jnp.maximum(prev, bm)

        @pl.when(i == _NI - 1)
        def _():
            tem_o[...] = tem_s[...]


def _pass2_kernel(pk_ref, whe_ref, se_ref, te_ref, tem_ref, out_o, acc, l):
    j = pl.program_id(1)

    @pl.when(j == 0)
    def _():
        acc[...] = jnp.zeros_like(acc)
        l[...] = jnp.zeros_like(l)

    pk = pk_ref[...].astype(jnp.int32)
    ok = jnp.concatenate([(pk >> r) & 1 for r in range(8)], axis=0) > 0
    rowv_j = (j * _BJ + jax.lax.broadcasted_iota(jnp.int32, (_BJ, 1), 0)) < _N
    whe = jnp.where(rowv_j, whe_ref[...], 0.0).astype(jnp.bfloat16)

    m = _lrelu(se_ref[...] + tem_ref[...])
    e = _lrelu(se_ref[...] + te_ref[...])
    p = jnp.where(ok, jnp.exp(e - m), 0.0)
    l[...] += jnp.sum(p, axis=1, keepdims=True)
    acc[...] += jnp.dot(p.astype(jnp.bfloat16), whe,
                        preferred_element_type=jnp.float32)

    @pl.when(j == _NJ - 1)
    def _():
        o = acc[...] / jnp.maximum(l[...], 1e-30)
        z = o - jnp.max(o, axis=1, keepdims=True)
        pz = jnp.exp(z)
        out_o[...] = pz / jnp.sum(pz, axis=1, keepdims=True)


def kernel(x, adj, W_lin, b_lin, W_heads, a_heads, W_end, a_end):
    f32 = jnp.float32
    w0, w1 = W_heads[0], W_heads[1]
    a01, a02 = a_heads[0, :_HID], a_heads[0, _HID:]
    a11, a12 = a_heads[1, :_HID], a_heads[1, _HID:]
    ae1, ae2 = a_end[:_OUT], a_end[_OUT:]
    blin = b_lin.reshape(1, _IN_F)

    const = lambda shape: pl.BlockSpec(shape, lambda *_: tuple(0 for _ in shape))

    wh0, wh1, s0, s1, t0, t1, t0m, t1m = pl.pallas_call(
        _prelude_kernel,
        grid=(_NI,),
        in_specs=[
            pl.BlockSpec((_BI, _IN_F), lambda i: (i, 0)),
            const((_IN_F, _IN_F)), const((1, _IN_F)),
            const((_IN_F, _HID)), const((_IN_F, _HID)),
            const((_HID, 1)), const((_HID, 1)),
            const((_HID, 1)), const((_HID, 1)),
        ],
        out_specs=[
            pl.BlockSpec((_BI, _HID), lambda i: (i, 0)),
            pl.BlockSpec((_BI, _HID), lambda i: (i, 0)),
            pl.BlockSpec((_BI, 1), lambda i: (i, 0)),
            pl.BlockSpec((_BI, 1), lambda i: (i, 0)),
            pl.BlockSpec((1, _BI), lambda i: (0, i)),
            pl.BlockSpec((1, _BI), lambda i: (0, i)),
            const((1, 1)), const((1, 1)),
        ],
        out_shape=[
            jax.ShapeDtypeStruct((_N, _HID), f32),
            jax.ShapeDtypeStruct((_N, _HID), f32),
            jax.ShapeDtypeStruct((_N, 1), f32),
            jax.ShapeDtypeStruct((_N, 1), f32),
            jax.ShapeDtypeStruct((1, _N), f32),
            jax.ShapeDtypeStruct((1, _N), f32),
            jax.ShapeDtypeStruct((1, 1), f32),
            jax.ShapeDtypeStruct((1, 1), f32),
        ],
        scratch_shapes=[pltpu.VMEM((1, 1), f32), pltpu.VMEM((1, 1), f32)],
    )(x, W_lin, blin, w0, w1, a01, a02, a11, a12)

    whe, se, te, tem, pk = pl.pallas_call(
        _pass1_kernel,
        grid=(_NI, _NJ),
        in_specs=[
            pl.BlockSpec((_BI, _BJ), lambda i, j: (i, j)),
            pl.BlockSpec((_BJ, _HID), lambda i, j: (j, 0)),
            pl.BlockSpec((_BJ, _HID), lambda i, j: (j, 0)),
            pl.BlockSpec((_BI, 1), lambda i, j: (i, 0)),
            pl.BlockSpec((_BI, 1), lambda i, j: (i, 0)),
            pl.BlockSpec((1, _BJ), lambda i, j: (0, j)),
            pl.BlockSpec((1, _BJ), lambda i, j: (0, j)),
            const((1, 1)), const((1, 1)),
            const((_IN_F, _OUT)),
            const((_OUT, 1)), const((_OUT, 1)),
        ],
        out_specs=[
            pl.BlockSpec((_BI, _OUT), lambda i, j: (i, 0)),
            pl.BlockSpec((_BI, 1), lambda i, j: (i, 0)),
            pl.BlockSpec((1, _BI), lambda i, j: (0, i)),
            const((1, 1)),
            pl.BlockSpec((_BG, _BJ), lambda i, j: (i, j)),
        ],
        out_shape=[
            jax.ShapeDtypeStruct((_N, _OUT), f32),
            jax.ShapeDtypeStruct((_N, 1), f32),
            jax.ShapeDtypeStruct((1, _N), f32),
            jax.ShapeDtypeStruct((1, 1), f32),
            jax.ShapeDtypeStruct((_NP, _NC), jnp.uint8),
        ],
        scratch_shapes=[
            pltpu.VMEM((_BI, _HID), f32), pltpu.VMEM((_BI, _HID), f32),
            pltpu.VMEM((_BI, 1), f32), pltpu.VMEM((_BI, 1), f32),
            pltpu.VMEM((1, 1), f32),
        ],
    )(adj, wh0, wh1, s0, s1, t0, t1, t0m, t1m, W_end, ae1, ae2)

    out = pl.pallas_call(
        _pass2_kernel,
        grid=(_NI, _NJ),
        in_specs=[
            pl.BlockSpec((_BG, _BJ), lambda i, j: (i, j)),
            pl.BlockSpec((_BJ, _OUT), lambda i, j: (j, 0)),
            pl.BlockSpec((_BI, 1), lambda i, j: (i, 0)),
            pl.BlockSpec((1, _BJ), lambda i, j: (0, j)),
            const((1, 1)),
        ],
        out_specs=pl.BlockSpec((_BI, _OUT), lambda i, j: (i, 0)),
        out_shape=jax.ShapeDtypeStruct((_N, _OUT), f32),
        scratch_shapes=[pltpu.VMEM((_BI, _OUT), f32), pltpu.VMEM((_BI, 1), f32)],
    )(pk, whe, se, te, tem)

    return out


# speedup vs baseline: 1.6148x; 1.6148x over previous
import jax
import jax.numpy as jnp
from jax.experimental import pallas as pl
from jax.experimental.pallas import tpu as pltpu

_N = 10000
_IN_F = 128
_HID = 64
_OUT = 64
_ALPHA = 0.2
_NEGBIG = -1e30
_LOG2E = 1.4426950408889634

_BI = 1024
_BJ = 1024
_NI = (_N + _BI - 1) // _BI
_NJ = (_N + _BJ - 1) // _BJ
_BG = _BI // 8
_NP = _NI * _BG
_NC = _NJ * _BJ
_EXT = 128

_f32 = jnp.float32
_bf16 = jnp.bfloat16


def _row_t(a2, wh):
    return jax.lax.dot_general(a2, wh, (((0,), (1,)), ((), ())),
                               preferred_element_type=_f32)


def _extend(wh, rowv):
    b = wh.shape[0]
    ext = jnp.concatenate(
        [wh, jnp.ones((b, 1), _f32), jnp.zeros((b, _EXT - _HID - 1), _f32)],
        axis=1)
    return jnp.where(rowv, ext, 0.0).astype(_bf16)


def _prep_st(wh, a1, a2, colv):
    s = jnp.dot(wh, a1, preferred_element_type=_f32) * _LOG2E
    t = _row_t(a2, wh) * _LOG2E
    tp = jnp.where(colv, t, _NEGBIG)
    tb = jnp.where(colv, t * _ALPHA, _NEGBIG)
    tm = jnp.max(jnp.where(colv, t, _NEGBIG), keepdims=True)
    return s, tp, tb, tm


def _lrelu(z):
    return jnp.maximum(z, _ALPHA * z)


def _prelude_kernel(x_ref, wlin_ref, blin_ref, w0_ref, w1_ref,
                    a01_ref, a02_ref, a11_ref, a12_ref,
                    wh0_o, wh1_o, s0_o, s1_o, t0p_o, t0b_o, t1p_o, t1b_o,
                    t0m_o, t1m_o, t0m_s, t1m_s):
    i = pl.program_id(0)
    h = jnp.dot(x_ref[...], wlin_ref[...],
                preferred_element_type=_f32) + blin_ref[...]
    wh0 = jnp.dot(h, w0_ref[...], preferred_element_type=_f32)
    wh1 = jnp.dot(h, w1_ref[...], preferred_element_type=_f32)
    rowv = (i * _BI + jax.lax.broadcasted_iota(jnp.int32, (_BI, 1), 0)) < _N
    colv = (i * _BI + jax.lax.broadcasted_iota(jnp.int32, (1, _BI), 1)) < _N
    wh0_o[...] = _extend(wh0, rowv)
    wh1_o[...] = _extend(wh1, rowv)
    s0, t0p, t0b, bm0 = _prep_st(wh0, a01_ref[...], a02_ref[...], colv)
    s1, t1p, t1b, bm1 = _prep_st(wh1, a11_ref[...], a12_ref[...], colv)
    s0_o[...] = s0
    s1_o[...] = s1
    t0p_o[...] = t0p
    t0b_o[...] = t0b
    t1p_o[...] = t1p
    t1b_o[...] = t1b
    prev0 = jnp.where(i == 0, jnp.full((1, 1), _NEGBIG), t0m_s[...])
    prev1 = jnp.where(i == 0, jnp.full((1, 1), _NEGBIG), t1m_s[...])
    t0m_s[...] = jnp.maximum(prev0, bm0)
    t1m_s[...] = jnp.maximum(prev1, bm1)

    @pl.when(i == _NI - 1)
    def _():
        t0m_o[...] = t0m_s[...]
        t1m_o[...] = t1m_s[...]


def _head_step(adjb, s_ref, tp_ref, tb_ref, tm_ref, wh_ref, acc):
    m = _lrelu(s_ref[...] + tm_ref[...])
    sa = s_ref[...] - m
    sb = s_ref[...] * _ALPHA - m
    q = jnp.maximum(sa + tp_ref[...], sb + tb_ref[...])
    x = jnp.exp2(q).astype(_bf16)
    p = jnp.where(adjb, x, _bf16(0.0))
    acc[...] += jnp.dot(p, wh_ref[...], preferred_element_type=_f32)


def _pass1_kernel(adj_ref, wh0_ref, wh1_ref, s0_ref, s1_ref,
                  t0p_ref, t0b_ref, t1p_ref, t1b_ref, t0m_ref, t1m_ref,
                  wend_ref, ae1_ref, ae2_ref,
                  whe_o, se_o, tep_o, teb_o, tem_o, pk_o,
                  acc0, acc1, tem_s):
    i = pl.program_id(0)
    j = pl.program_id(1)

    @pl.when(j == 0)
    def _():
        acc0[...] = jnp.zeros_like(acc0)
        acc1[...] = jnp.zeros_like(acc1)

    adjb = adj_ref[...] > 0

    oki = adjb.astype(jnp.int32)
    pk = oki[0:_BG, :]
    for r in range(1, 8):
        pk += oki[r * _BG:(r + 1) * _BG, :] << r
    pk_o[...] = pk.astype(jnp.uint8)

    _head_step(adjb, s0_ref, t0p_ref, t0b_ref, t0m_ref, wh0_ref, acc0)
    _head_step(adjb, s1_ref, t1p_ref, t1b_ref, t1m_ref, wh1_ref, acc1)

    @pl.when(j == _NJ - 1)
    def _():
        h0 = acc0[:, :_HID] / jnp.maximum(acc0[:, _HID:_HID + 1], 1e-30)
        h1 = acc1[:, :_HID] / jnp.maximum(acc1[:, _HID:_HID + 1], 1e-30)
        x0 = jnp.where(h0 > 0, h0, jnp.exp(h0) - 1.0)
        x1 = jnp.where(h1 > 0, h1, jnp.exp(h1) - 1.0)
        whe = (jnp.dot(x0, wend_ref[:_HID, :], preferred_element_type=_f32)
               + jnp.dot(x1, wend_ref[_HID:, :], preferred_element_type=_f32))
        rowv = (i * _BI + jax.lax.broadcasted_iota(jnp.int32, (_BI, 1), 0)) < _N
        colv = (i * _BI + jax.lax.broadcasted_iota(jnp.int32, (1, _BI), 1)) < _N
        whe_o[...] = _extend(whe, rowv)
        se, tep, teb, bm = _prep_st(whe, ae1_ref[...], ae2_ref[...], colv)
        se_o[...] = se
        tep_o[...] = tep
        teb_o[...] = teb
        prev = jnp.where(i == 0, jnp.full((1, 1), _NEGBIG), tem_s[...])
        tem_s[...] = jnp.maximum(prev, bm)

        @pl.when(i == _NI - 1)
        def _():
            tem_o[...] = tem_s[...]


def _pass2_kernel(pk_ref, whe_ref, se_ref, tep_ref, teb_ref, tem_ref,
                  out_o, acc):
    j = pl.program_id(1)

    @pl.when(j == 0)
    def _():
        acc[...] = jnp.zeros_like(acc)

    pk = pk_ref[...].astype(jnp.int32)
    adjb = jnp.concatenate([(pk >> r) & 1 for r in range(8)], axis=0) > 0
    _head_step(adjb, se_ref, tep_ref, teb_ref, tem_ref, whe_ref, acc)

    @pl.when(j == _NJ - 1)
    def _():
        o = acc[:, :_OUT] / jnp.maximum(acc[:, _OUT:_OUT + 1], 1e-30)
        z = o - jnp.max(o, axis=1, keepdims=True)
        pz = jnp.exp(z)
        out_o[...] = pz / jnp.sum(pz, axis=1, keepdims=True)


def kernel(x, adj, W_lin, b_lin, W_heads, a_heads, W_end, a_end):
    w0, w1 = W_heads[0], W_heads[1]
    a01, a02 = a_heads[0, :_HID], a_heads[0, _HID:]
    a11, a12 = a_heads[1, :_HID], a_heads[1, _HID:]
    ae1, ae2 = a_end[:_OUT], a_end[_OUT:]
    blin = b_lin.reshape(1, _IN_F)

    const = lambda shape: pl.BlockSpec(shape, lambda *_: tuple(0 for _ in shape))

    (wh0, wh1, s0, s1, t0p, t0b, t1p, t1b, t0m, t1m) = pl.pallas_call(
        _prelude_kernel,
        grid=(_NI,),
        in_specs=[
            pl.BlockSpec((_BI, _IN_F), lambda i: (i, 0)),
            const((_IN_F, _IN_F)), const((1, _IN_F)),
            const((_IN_F, _HID)), const((_IN_F, _HID)),
            const((_HID, 1)), const((_HID, 1)),
            const((_HID, 1)), const((_HID, 1)),
        ],
        out_specs=[
            pl.BlockSpec((_BI, _EXT), lambda i: (i, 0)),
            pl.BlockSpec((_BI, _EXT), lambda i: (i, 0)),
            pl.BlockSpec((_BI, 1), lambda i: (i, 0)),
            pl.BlockSpec((_BI, 1), lambda i: (i, 0)),
            pl.BlockSpec((1, _BI), lambda i: (0, i)),
            pl.BlockSpec((1, _BI), lambda i: (0, i)),
            pl.BlockSpec((1, _BI), lambda i: (0, i)),
            pl.BlockSpec((1, _BI), lambda i: (0, i)),
            const((1, 1)), const((1, 1)),
        ],
        out_shape=[
            jax.ShapeDtypeStruct((_NC, _EXT), _bf16),
            jax.ShapeDtypeStruct((_NC, _EXT), _bf16),
            jax.ShapeDtypeStruct((_NC, 1), _f32),
            jax.ShapeDtypeStruct((_NC, 1), _f32),
            jax.ShapeDtypeStruct((1, _NC), _f32),
            jax.ShapeDtypeStruct((1, _NC), _f32),
            jax.ShapeDtypeStruct((1, _NC), _f32),
            jax.ShapeDtypeStruct((1, _NC), _f32),
            jax.ShapeDtypeStruct((1, 1), _f32),
            jax.ShapeDtypeStruct((1, 1), _f32),
        ],
        scratch_shapes=[pltpu.VMEM((1, 1), _f32), pltpu.VMEM((1, 1), _f32)],
    )(x, W_lin, blin, w0, w1, a01, a02, a11, a12)

    whe, se, tep, teb, tem, pk = pl.pallas_call(
        _pass1_kernel,
        grid=(_NI, _NJ),
        in_specs=[
            pl.BlockSpec((_BI, _BJ), lambda i, j: (i, j)),
            pl.BlockSpec((_BJ, _EXT), lambda i, j: (j, 0)),
            pl.BlockSpec((_BJ, _EXT), lambda i, j: (j, 0)),
            pl.BlockSpec((_BI, 1), lambda i, j: (i, 0)),
            pl.BlockSpec((_BI, 1), lambda i, j: (i, 0)),
            pl.BlockSpec((1, _BJ), lambda i, j: (0, j)),
            pl.BlockSpec((1, _BJ), lambda i, j: (0, j)),
            pl.BlockSpec((1, _BJ), lambda i, j: (0, j)),
            pl.BlockSpec((1, _BJ), lambda i, j: (0, j)),
            const((1, 1)), const((1, 1)),
            const((_IN_F, _OUT)),
            const((_OUT, 1)), const((_OUT, 1)),
        ],
        out_specs=[
            pl.BlockSpec((_BI, _EXT), lambda i, j: (i, 0)),
            pl.BlockSpec((_BI, 1), lambda i, j: (i, 0)),
            pl.BlockSpec((1, _BI), lambda i, j: (0, i)),
            pl.BlockSpec((1, _BI), lambda i, j: (0, i)),
            const((1, 1)),
            pl.BlockSpec((_BG, _BJ), lambda i, j: (i, j)),
        ],
        out_shape=[
            jax.ShapeDtypeStruct((_NC, _EXT), _bf16),
            jax.ShapeDtypeStruct((_NC, 1), _f32),
            jax.ShapeDtypeStruct((1, _NC), _f32),
            jax.ShapeDtypeStruct((1, _NC), _f32),
            jax.ShapeDtypeStruct((1, 1), _f32),
            jax.ShapeDtypeStruct((_NP, _NC), jnp.uint8),
        ],
        scratch_shapes=[
            pltpu.VMEM((_BI, _EXT), _f32), pltpu.VMEM((_BI, _EXT), _f32),
            pltpu.VMEM((1, 1), _f32),
        ],
    )(adj, wh0, wh1, s0, s1, t0p, t0b, t1p, t1b, t0m, t1m, W_end, ae1, ae2)

    out = pl.pallas_call(
        _pass2_kernel,
        grid=(_NI, _NJ),
        in_specs=[
            pl.BlockSpec((_BG, _BJ), lambda i, j: (i, j)),
            pl.BlockSpec((_BJ, _EXT), lambda i, j: (j, 0)),
            pl.BlockSpec((_BI, 1), lambda i, j: (i, 0)),
            pl.BlockSpec((1, _BJ), lambda i, j: (0, j)),
            pl.BlockSpec((1, _BJ), lambda i, j: (0, j)),
            const((1, 1)),
        ],
        out_specs=pl.BlockSpec((_BI, _OUT), lambda i, j: (i, 0)),
        out_shape=jax.ShapeDtypeStruct((_N, _OUT), _f32),
        scratch_shapes=[pltpu.VMEM((_BI, _EXT), _f32)],
    )(pk, whe, se, tep, teb, tem)

    return out
